# Initial kernel scaffold; baseline (speedup 1.0000x reference)
import jax
import jax.numpy as jnp
from jax.experimental import pallas as pl


def kernel(x, perm, emb):
    raise NotImplementedError("write your pallas kernel here")


# speedup vs baseline: 1.9969x; 1.9969x over previous
import jax
import jax.numpy as jnp
from jax.experimental import pallas as pl

_N = 512
_P = 64
_E = 128


def _tree_sum_p(r):
    def tr16(u):
        u = u[0:8] + u[8:16]
        u = u[0:4] + u[4:8]
        u = u[0:2] + u[2:4]
        return u[0] + u[1]
    s0 = tr16(r[0:16])
    s1 = tr16(r[16:32])
    s2 = tr16(r[32:48])
    s3 = tr16(r[48:64])
    return ((s0 + s1) + s2) + s3


def _body(x_ref, permT_ref, emb_ref, out_ref):
    xb = x_ref[0]
    h, wb, p = xb.shape
    t_tok = h * wb
    xmin = jnp.min(xb, axis=0, keepdims=True)
    xmax = jnp.max(xb, axis=0, keepdims=True)
    den = (xmax - xmin) + jnp.float32(1e-8)
    xs = (xb - xmin) / den
    xs2 = xs.reshape(t_tok, p)
    xsT = xs2.T
    pT = permT_ref[...]
    r = pT[:, None, :] - xsT[:, :, None]
    t = _tree_sum_p(r)
    at = jnp.abs(t)
    m = jnp.min(at, axis=1, keepdims=True)
    ii = jax.lax.broadcasted_iota(jnp.int32, at.shape, 1)
    idx = jnp.min(jnp.where(at == m, ii, _N), axis=1)
    oh = (jax.lax.broadcasted_iota(jnp.int32, (t_tok, _N), 1)
          == idx[:, None]).astype(jnp.float32)
    ob = jax.lax.dot_general(oh, emb_ref[...],
                             (((1,), (0,)), ((), ())),
                             preferred_element_type=jnp.float32)
    out_ref[...] = ob.reshape(1, h, wb, _E)


def kernel(x, perm, emb):
    b, h, w, p = x.shape
    permT = perm.reshape(_N, _P).T
    grid = (b,)
    return pl.pallas_call(
        _body,
        grid=grid,
        in_specs=[
            pl.BlockSpec((1, h, w, p), lambda i: (i, 0, 0, 0)),
            pl.BlockSpec((_P, _N), lambda i: (0, 0)),
            pl.BlockSpec((_N, _E), lambda i: (0, 0)),
        ],
        out_specs=pl.BlockSpec((1, h, w, _E), lambda i: (i, 0, 0, 0)),
        out_shape=jax.ShapeDtypeStruct((b, h, w, _E), jnp.float32),
    )(x, permT, emb)
